# baseline (speedup 1.0000x reference)
import functools

import jax
import jax.numpy as jnp
from jax import lax
from jax.experimental import pallas as pl
from jax.experimental.pallas import tpu as pltpu
from jax.experimental.pallas import tpu_sc as plsc

N = 10000
E = 320000
D = 128
G = 128

NC = 2
NT = 16
CH = 64
IB = 16
NBLK = 20
RING = 4
E_PAD = NT * CH * IB * NBLK
RPT = 632
NP = NT * RPT

BN = 400
NB = N // BN


def _sc_agg_body(h_hbm, e_hbm, z_hbm, o1_hbm, o2_hbm,
                 agg_sh, s0, d0, s1, d1,
                 rows0, rows1, rows2, rows3,
                 isem0, isem1, rsem0, rsem1, rsem2, rsem3):
    c = lax.axis_index("c")
    s = lax.axis_index("s")
    base = s * RPT
    ring = [(rows0, rsem0), (rows1, rsem1), (rows2, rsem2), (rows3, rsem3)]

    pltpu.sync_copy(z_hbm, agg_sh.at[pl.ds(base, RPT)])
    plsc.subcore_barrier()

    def fetch_idx(k, sv, dv, isem):
        pltpu.async_copy(e_hbm.at[c, s, k, 0], sv, isem)
        pltpu.async_copy(e_hbm.at[c, s, k, 1], dv, isem)

    def wait_idx(k, sv, dv, isem):
        pltpu.make_async_copy(e_hbm.at[c, s, k, 0], sv, isem).wait()
        pltpu.make_async_copy(e_hbm.at[c, s, k, 1], dv, isem).wait()

    def process_block(sv, dv):
        for b in range(RING):
            buf, sem = ring[b]
            pltpu.async_copy(h_hbm.at[sv.at[b]], buf, sem)

        @pl.loop(0, IB, step=RING)
        def _steps(j):
            for b in range(RING):
                buf, sem = ring[b]
                pltpu.make_async_copy(h_hbm.at[sv.at[j + b]], buf, sem).wait()
                pltpu.sync_copy(buf, agg_sh.at[dv.at[j + b]], add=True)

                @pl.when(j + b + RING < IB)
                def _():
                    pltpu.async_copy(h_hbm.at[sv.at[j + b + RING]], buf, sem)

    fetch_idx(0, s0, d0, isem0)

    @pl.loop(0, NBLK, step=2)
    def _blocks(k):
        wait_idx(k, s0, d0, isem0)

        @pl.when(k + 1 < NBLK)
        def _():
            fetch_idx(k + 1, s1, d1, isem1)

        process_block(s0, d0)
        wait_idx(k + 1, s1, d1, isem1)

        @pl.when(k + 2 < NBLK)
        def _():
            fetch_idx(k + 2, s0, d0, isem0)

        process_block(s1, d1)

    plsc.subcore_barrier()

    @pl.when(c == 0)
    def _():
        pltpu.sync_copy(agg_sh.at[pl.ds(base, RPT)], o1_hbm.at[pl.ds(base, RPT)])

    @pl.when(c == 1)
    def _():
        pltpu.sync_copy(agg_sh.at[pl.ds(base, RPT)], o2_hbm.at[pl.ds(base, RPT)])


_sc_agg = pl.kernel(
    _sc_agg_body,
    out_type=(jax.ShapeDtypeStruct((NP, D), jnp.float32),
              jax.ShapeDtypeStruct((NP, D), jnp.float32)),
    mesh=plsc.VectorSubcoreMesh(core_axis_name="c", subcore_axis_name="s"),
    scratch_types=[
        pltpu.VMEM_SHARED((NP, D), jnp.float32),
        pltpu.VMEM((IB, CH), jnp.int32),
        pltpu.VMEM((IB, CH), jnp.int32),
        pltpu.VMEM((IB, CH), jnp.int32),
        pltpu.VMEM((IB, CH), jnp.int32),
        pltpu.VMEM((CH, D), jnp.float32),
        pltpu.VMEM((CH, D), jnp.float32),
        pltpu.VMEM((CH, D), jnp.float32),
        pltpu.VMEM((CH, D), jnp.float32),
        pltpu.SemaphoreType.DMA,
        pltpu.SemaphoreType.DMA,
        pltpu.SemaphoreType.DMA,
        pltpu.SemaphoreType.DMA,
        pltpu.SemaphoreType.DMA,
        pltpu.SemaphoreType.DMA,
    ],
)


def _tc_layer_body(h, a1, a2, w1a, b1a, w2a, b2a, w1b, b1b, w2b, b2b, o):
    x1 = h[...] + a1[...]
    z1 = jnp.maximum(lax.dot(x1, w1a[...],
                             preferred_element_type=jnp.float32) + b1a[...], 0.0)
    h1 = jnp.maximum(lax.dot(z1, w2a[...],
                             preferred_element_type=jnp.float32) + b2a[...], 0.0)
    x2 = h[...] + a2[...]
    z2 = jnp.maximum(lax.dot(x2, w1b[...],
                             preferred_element_type=jnp.float32) + b1b[...], 0.0)
    h2 = jnp.maximum(lax.dot(z2, w2b[...],
                             preferred_element_type=jnp.float32) + b2b[...], 0.0)
    o[...] = h1 + h2


def _tc_layer(h, a1, a2, wa, wb):
    row = pl.BlockSpec((BN, D), lambda i: (i, 0))
    wsp = pl.BlockSpec((D, D), lambda i: (0, 0))
    bsp = pl.BlockSpec((1, D), lambda i: (0, 0))
    return pl.pallas_call(
        _tc_layer_body,
        grid=(NB,),
        in_specs=[row, row, row, wsp, bsp, wsp, bsp, wsp, bsp, wsp, bsp],
        out_specs=row,
        out_shape=jax.ShapeDtypeStruct((N, D), jnp.float32),
    )(h, a1, a2, wa[0], wa[1], wa[2], wa[3], wb[0], wb[1], wb[2], wb[3])


def _tc_final_body(h, bat, fw1, fb1, fw2, fb2, o, acc):
    i = pl.program_id(0)

    @pl.when(i == 0)
    def _():
        acc[...] = jnp.zeros_like(acc)

    onehot = (bat[0] == lax.broadcasted_iota(jnp.int32, (BN, G), 1))
    onehot = onehot.astype(jnp.float32)
    acc[...] += lax.dot_general(onehot, h[...], (((0,), (0,)), ((), ())),
                                preferred_element_type=jnp.float32)

    @pl.when(i == NB - 1)
    def _():
        g = jnp.maximum(lax.dot(acc[...], fw1[...],
                                preferred_element_type=jnp.float32) + fb1[...], 0.0)
        o[...] = lax.dot(g, fw2[...],
                         preferred_element_type=jnp.float32) + fb2[...]


def _tc_final(h, batch_r, fp):
    row = pl.BlockSpec((BN, D), lambda i: (i, 0))
    bat = pl.BlockSpec((1, BN, 1), lambda i: (i, 0, 0))
    wsp = pl.BlockSpec((D, D), lambda i: (0, 0))
    bsp = pl.BlockSpec((1, D), lambda i: (0, 0))
    osp = pl.BlockSpec((G, D), lambda i: (0, 0))
    return pl.pallas_call(
        _tc_final_body,
        grid=(NB,),
        in_specs=[row, bat, wsp, bsp, wsp, bsp],
        out_specs=osp,
        out_shape=jax.ShapeDtypeStruct((G, D), jnp.float32),
        scratch_shapes=[pltpu.VMEM((G, D), jnp.float32)],
    )(h, batch_r, fp["W1"], fp["b1"].reshape(1, D), fp["W2"],
      fp["b2"].reshape(1, D))


def _fold_bn(p):
    s = p["gamma"] / jnp.sqrt(1.0 + 1e-5)
    w1 = p["W1"] * s[None, :]
    b1 = (p["b1"] * s + p["beta"]).reshape(1, D)
    return (w1, b1, p["W2"], p["b2"].reshape(1, D))


def kernel(x, edge_index, edge_index_diff, batch, params):
    e = jnp.stack([edge_index, edge_index_diff])
    pad = E_PAD - E
    src = jnp.pad(e[:, 0, :], ((0, 0), (0, pad))).reshape(2, NT, NBLK, IB, CH)
    dst = jnp.pad(e[:, 1, :], ((0, 0), (0, pad)),
                  constant_values=N).reshape(2, NT, NBLK, IB, CH)
    e_r = jnp.stack([src, dst], axis=3)
    zinit = jnp.zeros((RPT, D), jnp.float32)

    h = x
    for lp in params["layers"]:
        a1, a2 = _sc_agg(h, e_r, zinit)
        h = _tc_layer(h, a1, a2, _fold_bn(lp["conv"]), _fold_bn(lp["conv_diff"]))

    return _tc_final(h, batch.reshape(NB, BN, 1), params["final"])


# speedup vs baseline: 2.0641x; 2.0641x over previous
import jax
import jax.numpy as jnp
from jax import lax
from jax.experimental import pallas as pl
from jax.experimental.pallas import tpu as pltpu
from jax.experimental.pallas import tpu_sc as plsc

N = 10000
E = 320000
D = 128
DH = 64
G = 128

NC = 2
NT = 16
CH = 128
IB = 16
NBLK = 10
E_PAD = NT * CH * IB * NBLK
RING = 2
RPT = 632
NP = NT * RPT

BN = 400
NB = N // BN


def _sc_agg_body(h_hbm, e_hbm, z_hbm, o1_hbm, o2_hbm,
                 h_sh, agg_sh, s0, d0, s1, d1, rows0, rows1,
                 isem0, isem1, rsem0, rsem1):
    c = lax.axis_index("c")
    s = lax.axis_index("s")
    base = s * RPT
    ring = [(rows0, rsem0), (rows1, rsem1)]

    pltpu.sync_copy(h_hbm.at[c, pl.ds(base, RPT)], h_sh.at[pl.ds(base, RPT)])
    pltpu.sync_copy(z_hbm, agg_sh.at[pl.ds(base, RPT)])
    plsc.subcore_barrier()

    def fetch_idx(set_id, k, sv, dv, isem):
        pltpu.async_copy(e_hbm.at[s, set_id, k, 0], sv, isem)
        pltpu.async_copy(e_hbm.at[s, set_id, k, 1], dv, isem)

    def wait_idx(set_id, k, sv, dv, isem):
        pltpu.make_async_copy(e_hbm.at[s, set_id, k, 0], sv, isem).wait()
        pltpu.make_async_copy(e_hbm.at[s, set_id, k, 1], dv, isem).wait()

    def process_block(sv, dv):
        for b in range(RING):
            buf, sem = ring[b]
            pltpu.async_copy(h_sh.at[sv.at[b]], buf, sem)

        @pl.loop(0, IB, step=RING)
        def _steps(j):
            for b in range(RING):
                buf, sem = ring[b]
                pltpu.make_async_copy(h_sh.at[sv.at[j + b]], buf, sem).wait()
                pltpu.sync_copy(buf, agg_sh.at[dv.at[j + b]], add=True)

                @pl.when(j + b + RING < IB)
                def _():
                    pltpu.async_copy(h_sh.at[sv.at[j + b + RING]], buf, sem)

    for set_id, out_hbm in ((0, o1_hbm), (1, o2_hbm)):
        fetch_idx(set_id, 0, s0, d0, isem0)

        @pl.loop(0, NBLK, step=2)
        def _blocks(k):
            wait_idx(set_id, k, s0, d0, isem0)

            @pl.when(k + 1 < NBLK)
            def _():
                fetch_idx(set_id, k + 1, s1, d1, isem1)

            process_block(s0, d0)
            wait_idx(set_id, k + 1, s1, d1, isem1)

            @pl.when(k + 2 < NBLK)
            def _():
                fetch_idx(set_id, k + 2, s0, d0, isem0)

            process_block(s1, d1)

        plsc.subcore_barrier()
        pltpu.sync_copy(agg_sh.at[pl.ds(base, RPT)],
                        out_hbm.at[c, pl.ds(base, RPT)])
        if set_id == 0:
            pltpu.sync_copy(z_hbm, agg_sh.at[pl.ds(base, RPT)])
            plsc.subcore_barrier()


_sc_agg = pl.kernel(
    _sc_agg_body,
    out_type=(jax.ShapeDtypeStruct((NC, NP, DH), jnp.float32),
              jax.ShapeDtypeStruct((NC, NP, DH), jnp.float32)),
    mesh=plsc.VectorSubcoreMesh(core_axis_name="c", subcore_axis_name="s"),
    scratch_types=[
        pltpu.VMEM_SHARED((NP, DH), jnp.float32),
        pltpu.VMEM_SHARED((NP, DH), jnp.float32),
        pltpu.VMEM((IB, CH), jnp.int32),
        pltpu.VMEM((IB, CH), jnp.int32),
        pltpu.VMEM((IB, CH), jnp.int32),
        pltpu.VMEM((IB, CH), jnp.int32),
        pltpu.VMEM((CH, DH), jnp.float32),
        pltpu.VMEM((CH, DH), jnp.float32),
        pltpu.SemaphoreType.DMA,
        pltpu.SemaphoreType.DMA,
        pltpu.SemaphoreType.DMA,
        pltpu.SemaphoreType.DMA,
    ],
)


def _tc_layer_body(h, a1, a2, w1a, b1a, w2a, b2a, w1b, b1b, w2b, b2b, o):
    a1v = a1[...]
    a2v = a2[...]
    x1 = h[...] + jnp.concatenate([a1v[0], a1v[1]], axis=-1)
    z1 = jnp.maximum(lax.dot(x1, w1a[...],
                             preferred_element_type=jnp.float32) + b1a[...], 0.0)
    h1 = jnp.maximum(lax.dot(z1, w2a[...],
                             preferred_element_type=jnp.float32) + b2a[...], 0.0)
    x2 = h[...] + jnp.concatenate([a2v[0], a2v[1]], axis=-1)
    z2 = jnp.maximum(lax.dot(x2, w1b[...],
                             preferred_element_type=jnp.float32) + b1b[...], 0.0)
    h2 = jnp.maximum(lax.dot(z2, w2b[...],
                             preferred_element_type=jnp.float32) + b2b[...], 0.0)
    o[...] = h1 + h2


def _tc_layer(h, a1, a2, wa, wb):
    row = pl.BlockSpec((BN, D), lambda i: (i, 0))
    half = pl.BlockSpec((NC, BN, DH), lambda i: (0, i, 0))
    wsp = pl.BlockSpec((D, D), lambda i: (0, 0))
    bsp = pl.BlockSpec((1, D), lambda i: (0, 0))
    return pl.pallas_call(
        _tc_layer_body,
        grid=(NB,),
        in_specs=[row, half, half, wsp, bsp, wsp, bsp, wsp, bsp, wsp, bsp],
        out_specs=row,
        out_shape=jax.ShapeDtypeStruct((N, D), jnp.float32),
    )(h, a1, a2, wa[0], wa[1], wa[2], wa[3], wb[0], wb[1], wb[2], wb[3])


def _tc_final_body(h, bat, fw1, fb1, fw2, fb2, o, acc):
    i = pl.program_id(0)

    @pl.when(i == 0)
    def _():
        acc[...] = jnp.zeros_like(acc)

    onehot = (bat[0] == lax.broadcasted_iota(jnp.int32, (BN, G), 1))
    onehot = onehot.astype(jnp.float32)
    acc[...] += lax.dot_general(onehot, h[...], (((0,), (0,)), ((), ())),
                                preferred_element_type=jnp.float32)

    @pl.when(i == NB - 1)
    def _():
        g = jnp.maximum(lax.dot(acc[...], fw1[...],
                                preferred_element_type=jnp.float32) + fb1[...], 0.0)
        o[...] = lax.dot(g, fw2[...],
                         preferred_element_type=jnp.float32) + fb2[...]


def _tc_final(h, batch_r, fp):
    row = pl.BlockSpec((BN, D), lambda i: (i, 0))
    bat = pl.BlockSpec((1, BN, 1), lambda i: (i, 0, 0))
    wsp = pl.BlockSpec((D, D), lambda i: (0, 0))
    bsp = pl.BlockSpec((1, D), lambda i: (0, 0))
    osp = pl.BlockSpec((G, D), lambda i: (0, 0))
    return pl.pallas_call(
        _tc_final_body,
        grid=(NB,),
        in_specs=[row, bat, wsp, bsp, wsp, bsp],
        out_specs=osp,
        out_shape=jax.ShapeDtypeStruct((G, D), jnp.float32),
        scratch_shapes=[pltpu.VMEM((G, D), jnp.float32)],
    )(h, batch_r, fp["W1"], fp["b1"].reshape(1, D), fp["W2"],
      fp["b2"].reshape(1, D))


def _fold_bn(p):
    s = p["gamma"] / jnp.sqrt(1.0 + 1e-5)
    w1 = p["W1"] * s[None, :]
    b1 = (p["b1"] * s + p["beta"]).reshape(1, D)
    return (w1, b1, p["W2"], p["b2"].reshape(1, D))


def kernel(x, edge_index, edge_index_diff, batch, params):
    e = jnp.stack([edge_index, edge_index_diff])
    pad = E_PAD - E
    src = jnp.pad(e[:, 0, :], ((0, 0), (0, pad))).reshape(2, NT, NBLK, IB, CH)
    dst = jnp.pad(e[:, 1, :], ((0, 0), (0, pad)),
                  constant_values=N).reshape(2, NT, NBLK, IB, CH)
    e_r = jnp.stack([src, dst], axis=2).transpose(1, 0, 3, 2, 4, 5)
    zinit = jnp.zeros((RPT, DH), jnp.float32)

    h = x
    for lp in params["layers"]:
        hh = jnp.pad(h, ((0, NP - N), (0, 0)))
        hh = hh.reshape(NP, NC, DH).transpose(1, 0, 2)
        a1, a2 = _sc_agg(hh, e_r, zinit)
        h = _tc_layer(h, a1, a2, _fold_bn(lp["conv"]), _fold_bn(lp["conv_diff"]))

    return _tc_final(h, batch.reshape(NB, BN, 1), params["final"])
